# Initial kernel scaffold; baseline (speedup 1.0000x reference)
import jax
import jax.numpy as jnp
from jax.experimental import pallas as pl


def kernel(x, w_gate, b_gate, gain, bias, log_mix, log_scale):
    raise NotImplementedError("write your pallas kernel here")


# speedup vs baseline: 7.6855x; 7.6855x over previous
import functools

import jax
import jax.numpy as jnp
from jax import lax
from jax.experimental import pallas as pl
from jax.experimental.pallas import tpu as pltpu
from jax.experimental.pallas import tpu_sc as plsc

T = 4096
D = 768
K = 8
RB = 256
CB = 256
NEG = float(-3.4e38)
INVALID_THRESH = float(-1e30)

_NC = 2
_NS = 16
_NW = _NC * _NS
_TOK_PER_W = T // _NW
_CH_TOK = 8
_CH = _CH_TOK * K
_STEPS = _TOK_PER_W // _CH_TOK


def _topk_kernel(x_blk_ref, x_all_ref, idx_out_ref):
    i = pl.program_id(0)
    x_r = x_blk_ref[...]
    row_g = i * RB + lax.broadcasted_iota(jnp.int32, (RB, CB), 0)
    col_l = lax.broadcasted_iota(jnp.int32, (RB, CB), 1)
    pos_big = jnp.int32(2 ** 30)

    vals0 = jnp.full((RB, K), NEG, jnp.float32)
    idx0 = jnp.zeros((RB, K), jnp.int32)

    def body(ci, carry):
        vals, idx = carry
        x_c = x_all_ref[pl.ds(ci * CB, CB), :]
        s = lax.dot_general(
            x_r, x_c, (((1,), (1,)), ((), ())),
            preferred_element_type=jnp.float32)
        col_g = ci * CB + col_l
        s = jnp.where(col_g <= row_g, s, NEG)
        cv = jnp.concatenate([vals, s], axis=1)
        cidx = jnp.concatenate([idx, col_g], axis=1)
        nv = []
        ni = []
        for _ in range(K):
            m_val = jnp.max(cv, axis=1, keepdims=True)
            m_idx = jnp.min(
                jnp.where(cv == m_val, cidx, pos_big), axis=1, keepdims=True)
            nv.append(m_val)
            ni.append(m_idx)
            cv = jnp.where((cv == m_val) & (cidx == m_idx), NEG, cv)
        return (jnp.concatenate(nv, axis=1), jnp.concatenate(ni, axis=1))

    vals, idx = lax.fori_loop(0, i + 1, body, (vals0, idx0))
    idx_out_ref[...] = jnp.where(vals <= INVALID_THRESH, jnp.int32(T), idx)


def _topk_indices(x2d):
    return pl.pallas_call(
        _topk_kernel,
        grid=(T // RB,),
        in_specs=[
            pl.BlockSpec((RB, D), lambda i: (i, 0)),
            pl.BlockSpec((T, D), lambda i: (0, 0)),
        ],
        out_specs=pl.BlockSpec((RB, K), lambda i: (i, 0)),
        out_shape=jax.ShapeDtypeStruct((T, K), jnp.int32),
        compiler_params=pltpu.CompilerParams(
            dimension_semantics=("arbitrary",)),
    )(x2d, x2d)


def _gather_sum(x_pad, idx_flat):
    mesh = plsc.VectorSubcoreMesh(core_axis_name="c", subcore_axis_name="s")

    @functools.partial(
        pl.kernel,
        mesh=mesh,
        out_type=jax.ShapeDtypeStruct((T, D), jnp.float32),
        scratch_types=[
            pltpu.VMEM((_CH,), jnp.int32),
            pltpu.VMEM((_CH, D), jnp.float32),
            pltpu.VMEM((_CH_TOK, D), jnp.float32),
            pltpu.SemaphoreType.DMA,
        ],
    )
    def k(table_hbm, idx_hbm, out_hbm, idx_v, rows_v, out_v, sem):
        wid = lax.axis_index("s") * _NC + lax.axis_index("c")
        idx_base = wid * _TOK_PER_W * K
        tok_base = wid * _TOK_PER_W

        @pl.loop(0, _STEPS)
        def _(it):
            pltpu.sync_copy(idx_hbm.at[pl.ds(idx_base + it * _CH, _CH)], idx_v)
            pltpu.async_copy(table_hbm.at[idx_v], rows_v, sem).wait()

            @pl.loop(0, _CH_TOK)
            def _(g):
                @pl.loop(0, D, step=16)
                def _(c):
                    acc = rows_v[K * g, pl.ds(c, 16)]
                    for r in range(1, K):
                        acc = acc + rows_v[K * g + r, pl.ds(c, 16)]
                    out_v[g, pl.ds(c, 16)] = acc

            pltpu.sync_copy(
                out_v, out_hbm.at[pl.ds(tok_base + it * _CH_TOK, _CH_TOK)])

    return k(x_pad, idx_flat)


def _tail_kernel(x_ref, ms_ref, wg_ref, gain_ref, bias_ref, sc_ref, out_ref):
    i = pl.program_id(0)
    x = x_ref[...]
    msum = ms_ref[...]
    w_gate = wg_ref[...]
    gain = gain_ref[...]
    bias = bias_ref[...]
    b_gate = sc_ref[0, 0]
    log_mix = sc_ref[0, 1]
    log_scale = sc_ref[0, 2]

    mix = jax.nn.sigmoid(log_mix)
    scale = jax.nn.softplus(log_scale) + jnp.float32(0.01)

    t = i * RB + lax.broadcasted_iota(jnp.int32, (RB, 1), 0)
    deg = jnp.minimum((t + 1).astype(jnp.float32), jnp.float32(K))
    msg = msum / deg

    gate_logit = jnp.sum(x * w_gate, axis=1, keepdims=True) + b_gate
    gate = jax.nn.sigmoid(gate_logit)
    blended = mix * x + (1.0 - mix) * msg
    z = blended * gain + bias
    gelu = 0.5 * z * (1.0 + lax.erf(z * jnp.float32(0.7071067811865476)))
    delta_raw = gelu * scale
    out_ref[...] = gate * delta_raw


def _tail(x2d, msum, w_gate, gain, bias, scalars):
    return pl.pallas_call(
        _tail_kernel,
        grid=(T // RB,),
        in_specs=[
            pl.BlockSpec((RB, D), lambda i: (i, 0)),
            pl.BlockSpec((RB, D), lambda i: (i, 0)),
            pl.BlockSpec((1, D), lambda i: (0, 0)),
            pl.BlockSpec((1, D), lambda i: (0, 0)),
            pl.BlockSpec((1, D), lambda i: (0, 0)),
            pl.BlockSpec((1, 4), lambda i: (0, 0)),
        ],
        out_specs=pl.BlockSpec((RB, D), lambda i: (i, 0)),
        out_shape=jax.ShapeDtypeStruct((T, D), jnp.float32),
        compiler_params=pltpu.CompilerParams(
            dimension_semantics=("arbitrary",)),
    )(x2d, msum, w_gate, gain, bias, scalars)


@jax.jit
def kernel(x, w_gate, b_gate, gain, bias, log_mix, log_scale):
    x2d = x[0]
    idx = _topk_indices(x2d)
    x_pad = jnp.concatenate(
        [x2d, jnp.zeros((8, D), jnp.float32)], axis=0)
    msum = _gather_sum(x_pad, idx.reshape(T * K))
    scalars = jnp.stack(
        [b_gate, log_mix, log_scale, jnp.float32(0.0)]).reshape(1, 4)
    delta = _tail(x2d, msum, w_gate.reshape(1, D), gain.reshape(1, D),
                  bias.reshape(1, D), scalars)
    return delta[None]


# speedup vs baseline: 11.6967x; 1.5219x over previous
import functools

import jax
import jax.numpy as jnp
from jax import lax
from jax.experimental import pallas as pl
from jax.experimental.pallas import tpu as pltpu
from jax.experimental.pallas import tpu_sc as plsc

T = 4096
D = 768
K = 8
RB = 256
CB = 256
NEG = float(-3.4e38)
INVALID_THRESH = float(-1e30)

_NC = 2
_NS = 16
_NW = _NC * _NS
_TOK_PER_W = T // _NW
_CH_TOK = 8
_CH = _CH_TOK * K
_STEPS = _TOK_PER_W // _CH_TOK


def _topk_kernel(x_blk_ref, x_all_ref, idx_out_ref, M_ref, Mi_ref):
    i = pl.program_id(0)
    x_r = x_blk_ref[...]
    row_g = i * RB + lax.broadcasted_iota(jnp.int32, (RB, 128), 0)
    lane = lax.broadcasted_iota(jnp.int32, (RB, 128), 1)
    pos_big = jnp.int32(2 ** 30)

    for m in range(K):
        M_ref[m] = jnp.full((RB, 128), NEG, jnp.float32)
        Mi_ref[m] = jnp.zeros((RB, 128), jnp.int32)

    def body(ci, _):
        x_c = x_all_ref[pl.ds(ci * CB, CB), :]
        s = lax.dot_general(
            x_r, x_c, (((1,), (1,)), ((), ())),
            preferred_element_type=jnp.float32)
        for sl in range(CB // 128):
            v = s[:, sl * 128:(sl + 1) * 128]
            col = ci * CB + sl * 128 + lane
            v = jnp.where(col <= row_g, v, NEG)
            ge = [M_ref[m] >= v for m in range(K)]
            for m in range(K - 1, 0, -1):
                M_ref[m] = jnp.where(
                    ge[m], M_ref[m], jnp.where(ge[m - 1], v, M_ref[m - 1]))
                Mi_ref[m] = jnp.where(
                    ge[m], Mi_ref[m], jnp.where(ge[m - 1], col, Mi_ref[m - 1]))
            M_ref[0] = jnp.where(ge[0], M_ref[0], v)
            Mi_ref[0] = jnp.where(ge[0], Mi_ref[0], col)
        return 0

    lax.fori_loop(0, i + 1, body, 0)

    vals_out = []
    idx_out = []
    for _ in range(K):
        top = M_ref[0]
        topi = Mi_ref[0]
        m_val = jnp.max(top, axis=1, keepdims=True)
        m_idx = jnp.min(
            jnp.where(top == m_val, topi, pos_big), axis=1, keepdims=True)
        vals_out.append(m_val)
        idx_out.append(m_idx)
        lanemask = (top == m_val) & (topi == m_idx)
        for m in range(K - 1):
            M_ref[m] = jnp.where(lanemask, M_ref[m + 1], M_ref[m])
            Mi_ref[m] = jnp.where(lanemask, Mi_ref[m + 1], Mi_ref[m])
        M_ref[K - 1] = jnp.where(lanemask, NEG, M_ref[K - 1])

    vals = jnp.concatenate(vals_out, axis=1)
    idx = jnp.concatenate(idx_out, axis=1)
    idx_out_ref[...] = jnp.where(vals <= INVALID_THRESH, jnp.int32(T), idx)


def _topk_indices(x2d):
    return pl.pallas_call(
        _topk_kernel,
        grid=(T // RB,),
        in_specs=[
            pl.BlockSpec((RB, D), lambda i: (i, 0)),
            pl.BlockSpec((T, D), lambda i: (0, 0)),
        ],
        out_specs=pl.BlockSpec((RB, K), lambda i: (i, 0)),
        out_shape=jax.ShapeDtypeStruct((T, K), jnp.int32),
        scratch_shapes=[
            pltpu.VMEM((K, RB, 128), jnp.float32),
            pltpu.VMEM((K, RB, 128), jnp.int32),
        ],
        compiler_params=pltpu.CompilerParams(
            dimension_semantics=("arbitrary",)),
    )(x2d, x2d)


def _gather_sum(x_pad, idx_flat):
    mesh = plsc.VectorSubcoreMesh(core_axis_name="c", subcore_axis_name="s")

    @functools.partial(
        pl.kernel,
        mesh=mesh,
        out_type=jax.ShapeDtypeStruct((T, D), jnp.float32),
        scratch_types=[
            pltpu.VMEM((_CH,), jnp.int32),
            pltpu.VMEM((_CH, D), jnp.float32),
            pltpu.VMEM((_CH_TOK, D), jnp.float32),
            pltpu.SemaphoreType.DMA,
        ],
    )
    def k(table_hbm, idx_hbm, out_hbm, idx_v, rows_v, out_v, sem):
        wid = lax.axis_index("s") * _NC + lax.axis_index("c")
        idx_base = wid * _TOK_PER_W * K
        tok_base = wid * _TOK_PER_W

        @pl.loop(0, _STEPS)
        def _(it):
            pltpu.sync_copy(idx_hbm.at[pl.ds(idx_base + it * _CH, _CH)], idx_v)
            pltpu.async_copy(table_hbm.at[idx_v], rows_v, sem).wait()

            @pl.loop(0, _CH_TOK)
            def _(g):
                @pl.loop(0, D, step=16)
                def _(c):
                    acc = rows_v[K * g, pl.ds(c, 16)]
                    for r in range(1, K):
                        acc = acc + rows_v[K * g + r, pl.ds(c, 16)]
                    out_v[g, pl.ds(c, 16)] = acc

            pltpu.sync_copy(
                out_v, out_hbm.at[pl.ds(tok_base + it * _CH_TOK, _CH_TOK)])

    return k(x_pad, idx_flat)


def _tail_kernel(x_ref, ms_ref, wg_ref, gain_ref, bias_ref, sc_ref, out_ref):
    i = pl.program_id(0)
    x = x_ref[...]
    msum = ms_ref[...]
    w_gate = wg_ref[...]
    gain = gain_ref[...]
    bias = bias_ref[...]
    b_gate = sc_ref[0, 0]
    log_mix = sc_ref[0, 1]
    log_scale = sc_ref[0, 2]

    mix = jax.nn.sigmoid(log_mix)
    scale = jax.nn.softplus(log_scale) + jnp.float32(0.01)

    t = i * RB + lax.broadcasted_iota(jnp.int32, (RB, 1), 0)
    deg = jnp.minimum((t + 1).astype(jnp.float32), jnp.float32(K))
    msg = msum / deg

    gate_logit = jnp.sum(x * w_gate, axis=1, keepdims=True) + b_gate
    gate = jax.nn.sigmoid(gate_logit)
    blended = mix * x + (1.0 - mix) * msg
    z = blended * gain + bias
    gelu = 0.5 * z * (1.0 + lax.erf(z * jnp.float32(0.7071067811865476)))
    delta_raw = gelu * scale
    out_ref[...] = gate * delta_raw


def _tail(x2d, msum, w_gate, gain, bias, scalars):
    return pl.pallas_call(
        _tail_kernel,
        grid=(T // RB,),
        in_specs=[
            pl.BlockSpec((RB, D), lambda i: (i, 0)),
            pl.BlockSpec((RB, D), lambda i: (i, 0)),
            pl.BlockSpec((1, D), lambda i: (0, 0)),
            pl.BlockSpec((1, D), lambda i: (0, 0)),
            pl.BlockSpec((1, D), lambda i: (0, 0)),
            pl.BlockSpec((1, 4), lambda i: (0, 0)),
        ],
        out_specs=pl.BlockSpec((RB, D), lambda i: (i, 0)),
        out_shape=jax.ShapeDtypeStruct((T, D), jnp.float32),
        compiler_params=pltpu.CompilerParams(
            dimension_semantics=("arbitrary",)),
    )(x2d, msum, w_gate, gain, bias, scalars)


@jax.jit
def kernel(x, w_gate, b_gate, gain, bias, log_mix, log_scale):
    x2d = x[0]
    idx = _topk_indices(x2d)
    x_pad = jnp.concatenate(
        [x2d, jnp.zeros((8, D), jnp.float32)], axis=0)
    msum = _gather_sum(x_pad, idx.reshape(T * K))
    scalars = jnp.stack(
        [b_gate, log_mix, log_scale, jnp.float32(0.0)]).reshape(1, 4)
    delta = _tail(x2d, msum, w_gate.reshape(1, D), gain.reshape(1, D),
                  bias.reshape(1, D), scalars)
    return delta[None]


# speedup vs baseline: 13.0920x; 1.1193x over previous
import functools

import jax
import jax.numpy as jnp
from jax import lax
from jax.experimental import pallas as pl
from jax.experimental.pallas import tpu as pltpu
from jax.experimental.pallas import tpu_sc as plsc

T = 4096
D = 768
K = 8
RB = 256
CB = 256
NEG = float(-3.4e38)
INVALID_THRESH = float(-1e30)

_NC = 2
_NS = 16
_NW = _NC * _NS
_TOK_PER_W = T // _NW
_CH_TOK = 8
_CH = _CH_TOK * K
_STEPS = _TOK_PER_W // _CH_TOK


def _topk_kernel(x_blk_ref, x_all_ref, idx_out_ref, M_ref, Mi_ref):
    i = pl.program_id(0)
    x_r = x_blk_ref[...]
    row_g = i * RB + lax.broadcasted_iota(jnp.int32, (RB, 128), 0)
    lane = lax.broadcasted_iota(jnp.int32, (RB, 128), 1)
    pos_big = jnp.int32(2 ** 30)

    for m in range(K):
        M_ref[m] = jnp.full((RB, 128), NEG, jnp.float32)
        Mi_ref[m] = jnp.zeros((RB, 128), jnp.int32)

    def body(ci, _):
        x_c = x_all_ref[pl.ds(ci * CB, CB), :]
        s = lax.dot_general(
            x_r, x_c, (((1,), (1,)), ((), ())),
            preferred_element_type=jnp.float32)
        for sl in range(CB // 128):
            v = s[:, sl * 128:(sl + 1) * 128]
            col = ci * CB + sl * 128 + lane
            v = jnp.where(col <= row_g, v, NEG)
            ge = [M_ref[m] >= v for m in range(K)]
            for m in range(K - 1, 0, -1):
                M_ref[m] = jnp.where(
                    ge[m], M_ref[m], jnp.where(ge[m - 1], v, M_ref[m - 1]))
                Mi_ref[m] = jnp.where(
                    ge[m], Mi_ref[m], jnp.where(ge[m - 1], col, Mi_ref[m - 1]))
            M_ref[0] = jnp.where(ge[0], M_ref[0], v)
            Mi_ref[0] = jnp.where(ge[0], Mi_ref[0], col)
        return 0

    lax.fori_loop(0, i + 1, body, 0)

    vals_out = []
    idx_out = []
    for _ in range(K):
        top = M_ref[0]
        topi = Mi_ref[0]
        m_val = jnp.max(top, axis=1, keepdims=True)
        m_idx = jnp.min(
            jnp.where(top == m_val, topi, pos_big), axis=1, keepdims=True)
        vals_out.append(m_val)
        idx_out.append(m_idx)
        lanemask = (top == m_val) & (topi == m_idx)
        for m in range(K - 1):
            M_ref[m] = jnp.where(lanemask, M_ref[m + 1], M_ref[m])
            Mi_ref[m] = jnp.where(lanemask, Mi_ref[m + 1], Mi_ref[m])
        M_ref[K - 1] = jnp.where(lanemask, NEG, M_ref[K - 1])

    vals = jnp.concatenate(vals_out, axis=1)
    idx = jnp.concatenate(idx_out, axis=1)
    idx_out_ref[...] = jnp.where(vals <= INVALID_THRESH, jnp.int32(T), idx)


def _topk_indices(x2d):
    return pl.pallas_call(
        _topk_kernel,
        grid=(T // RB,),
        in_specs=[
            pl.BlockSpec((RB, D), lambda i: (i, 0)),
            pl.BlockSpec((T, D), lambda i: (0, 0)),
        ],
        out_specs=pl.BlockSpec((RB, K), lambda i: (i, 0)),
        out_shape=jax.ShapeDtypeStruct((T, K), jnp.int32),
        scratch_shapes=[
            pltpu.VMEM((K, RB, 128), jnp.float32),
            pltpu.VMEM((K, RB, 128), jnp.int32),
        ],
        compiler_params=pltpu.CompilerParams(
            dimension_semantics=("arbitrary",)),
    )(x2d, x2d)


def _gather_sum(x_pad, idx_flat):
    mesh = plsc.VectorSubcoreMesh(core_axis_name="c", subcore_axis_name="s")

    @functools.partial(
        pl.kernel,
        mesh=mesh,
        out_type=jax.ShapeDtypeStruct((T, D), jnp.float32),
        scratch_types=[
            pltpu.VMEM((_CH,), jnp.int32),
            pltpu.VMEM((_CH,), jnp.int32),
            pltpu.VMEM((_CH, D), jnp.float32),
            pltpu.VMEM((_CH, D), jnp.float32),
            pltpu.VMEM((_CH_TOK, D), jnp.float32),
            pltpu.SemaphoreType.DMA,
            pltpu.SemaphoreType.DMA,
        ],
    )
    def k(table_hbm, idx_hbm, out_hbm, idx_v0, idx_v1, rows_v0, rows_v1,
          out_v, sem0, sem1):
        wid = lax.axis_index("s") * _NC + lax.axis_index("c")
        idx_base = wid * _TOK_PER_W * K
        tok_base = wid * _TOK_PER_W
        bufs = ((idx_v0, rows_v0, sem0), (idx_v1, rows_v1, sem1))

        def start(b, it):
            idx_v, rows_v, sem = bufs[b]
            pltpu.sync_copy(idx_hbm.at[pl.ds(idx_base + it * _CH, _CH)], idx_v)
            pltpu.make_async_copy(table_hbm.at[idx_v], rows_v, sem).start()

        def consume(b, it):
            idx_v, rows_v, sem = bufs[b]
            pltpu.make_async_copy(table_hbm.at[idx_v], rows_v, sem).wait()

            @pl.loop(0, _CH_TOK)
            def _(g):
                @pl.loop(0, D, step=16)
                def _(c):
                    acc = rows_v[K * g, pl.ds(c, 16)]
                    for r in range(1, K):
                        acc = acc + rows_v[K * g + r, pl.ds(c, 16)]
                    out_v[g, pl.ds(c, 16)] = acc

            pltpu.sync_copy(
                out_v, out_hbm.at[pl.ds(tok_base + it * _CH_TOK, _CH_TOK)])

        start(0, 0)

        @pl.loop(0, _STEPS, step=2)
        def _(it):
            start(1, it + 1)
            consume(0, it)

            @pl.when(it + 2 < _STEPS)
            def _():
                start(0, it + 2)

            consume(1, it + 1)

    return k(x_pad, idx_flat)


def _tail_kernel(x_ref, ms_ref, wg_ref, gain_ref, bias_ref, sc_ref, out_ref):
    i = pl.program_id(0)
    x = x_ref[...]
    msum = ms_ref[...]
    w_gate = wg_ref[...]
    gain = gain_ref[...]
    bias = bias_ref[...]
    b_gate = sc_ref[0, 0]
    log_mix = sc_ref[0, 1]
    log_scale = sc_ref[0, 2]

    mix = jax.nn.sigmoid(log_mix)
    scale = jax.nn.softplus(log_scale) + jnp.float32(0.01)

    t = i * RB + lax.broadcasted_iota(jnp.int32, (RB, 1), 0)
    deg = jnp.minimum((t + 1).astype(jnp.float32), jnp.float32(K))
    msg = msum / deg

    gate_logit = jnp.sum(x * w_gate, axis=1, keepdims=True) + b_gate
    gate = jax.nn.sigmoid(gate_logit)
    blended = mix * x + (1.0 - mix) * msg
    z = blended * gain + bias
    gelu = 0.5 * z * (1.0 + lax.erf(z * jnp.float32(0.7071067811865476)))
    delta_raw = gelu * scale
    out_ref[...] = gate * delta_raw


def _tail(x2d, msum, w_gate, gain, bias, scalars):
    return pl.pallas_call(
        _tail_kernel,
        grid=(T // RB,),
        in_specs=[
            pl.BlockSpec((RB, D), lambda i: (i, 0)),
            pl.BlockSpec((RB, D), lambda i: (i, 0)),
            pl.BlockSpec((1, D), lambda i: (0, 0)),
            pl.BlockSpec((1, D), lambda i: (0, 0)),
            pl.BlockSpec((1, D), lambda i: (0, 0)),
            pl.BlockSpec((1, 4), lambda i: (0, 0)),
        ],
        out_specs=pl.BlockSpec((RB, D), lambda i: (i, 0)),
        out_shape=jax.ShapeDtypeStruct((T, D), jnp.float32),
        compiler_params=pltpu.CompilerParams(
            dimension_semantics=("arbitrary",)),
    )(x2d, msum, w_gate, gain, bias, scalars)


@jax.jit
def kernel(x, w_gate, b_gate, gain, bias, log_mix, log_scale):
    x2d = x[0]
    idx = _topk_indices(x2d)
    x_pad = jnp.concatenate(
        [x2d, jnp.zeros((8, D), jnp.float32)], axis=0)
    msum = _gather_sum(x_pad, idx.reshape(T * K))
    scalars = jnp.stack(
        [b_gate, log_mix, log_scale, jnp.float32(0.0)]).reshape(1, 4)
    delta = _tail(x2d, msum, w_gate.reshape(1, D), gain.reshape(1, D),
                  bias.reshape(1, D), scalars)
    return delta[None]


# speedup vs baseline: 14.5214x; 1.1092x over previous
import functools

import jax
import jax.numpy as jnp
from jax import lax
from jax.experimental import pallas as pl
from jax.experimental.pallas import tpu as pltpu
from jax.experimental.pallas import tpu_sc as plsc

T = 4096
D = 768
K = 8
RB = 256
CB = 256
NB = T // RB
NEG = float(-3.4e38)
INVALID_THRESH = float(-1e30)

_NC = 2
_NS = 16
_NW = _NC * _NS
_CH_TOK = 8
_CH = _CH_TOK * K


def _topk_kernel(x_blk_ref, x_all_ref, idx_out_ref, M_ref, Mi_ref, *, blk_lo):
    i = pl.program_id(0) + blk_lo
    x_r = x_blk_ref[...]
    row_g = i * RB + lax.broadcasted_iota(jnp.int32, (RB, 128), 0)
    lane = lax.broadcasted_iota(jnp.int32, (RB, 128), 1)
    pos_big = jnp.int32(2 ** 30)

    for m in range(K):
        M_ref[m] = jnp.full((RB, 128), NEG, jnp.float32)
        Mi_ref[m] = jnp.zeros((RB, 128), jnp.int32)

    def body(ci, _):
        x_c = x_all_ref[pl.ds(ci * CB, CB), :]
        s = lax.dot_general(
            x_r, x_c, (((1,), (1,)), ((), ())),
            preferred_element_type=jnp.float32)
        for sl in range(CB // 128):
            v = s[:, sl * 128:(sl + 1) * 128]
            col = ci * CB + sl * 128 + lane
            v = jnp.where(col <= row_g, v, NEG)
            ge = [M_ref[m] >= v for m in range(K)]
            for m in range(K - 1, 0, -1):
                M_ref[m] = jnp.where(
                    ge[m], M_ref[m], jnp.where(ge[m - 1], v, M_ref[m - 1]))
                Mi_ref[m] = jnp.where(
                    ge[m], Mi_ref[m], jnp.where(ge[m - 1], col, Mi_ref[m - 1]))
            M_ref[0] = jnp.where(ge[0], M_ref[0], v)
            Mi_ref[0] = jnp.where(ge[0], Mi_ref[0], col)
        return 0

    lax.fori_loop(0, i + 1, body, 0)

    vals_out = []
    idx_out = []
    for _ in range(K):
        top = M_ref[0]
        topi = Mi_ref[0]
        m_val = jnp.max(top, axis=1, keepdims=True)
        m_idx = jnp.min(
            jnp.where(top == m_val, topi, pos_big), axis=1, keepdims=True)
        vals_out.append(m_val)
        idx_out.append(m_idx)
        lanemask = (top == m_val) & (topi == m_idx)
        for m in range(K - 1):
            M_ref[m] = jnp.where(lanemask, M_ref[m + 1], M_ref[m])
            Mi_ref[m] = jnp.where(lanemask, Mi_ref[m + 1], Mi_ref[m])
        M_ref[K - 1] = jnp.where(lanemask, NEG, M_ref[K - 1])

    vals = jnp.concatenate(vals_out, axis=1)
    idx = jnp.concatenate(idx_out, axis=1)
    idx_out_ref[...] = jnp.where(vals <= INVALID_THRESH, jnp.int32(T), idx)


def _topk_indices(x2d, blk_lo, n_blk):
    return pl.pallas_call(
        functools.partial(_topk_kernel, blk_lo=blk_lo),
        grid=(n_blk,),
        in_specs=[
            pl.BlockSpec((RB, D), lambda i: (i + blk_lo, 0)),
            pl.BlockSpec((T, D), lambda i: (0, 0)),
        ],
        out_specs=pl.BlockSpec((RB, K), lambda i: (i, 0)),
        out_shape=jax.ShapeDtypeStruct((n_blk * RB, K), jnp.int32),
        scratch_shapes=[
            pltpu.VMEM((K, RB, 128), jnp.float32),
            pltpu.VMEM((K, RB, 128), jnp.int32),
        ],
        compiler_params=pltpu.CompilerParams(
            dimension_semantics=("arbitrary",)),
    )(x2d, x2d)


def _gather_sum(x_pad, idx_flat, n_tok):
    mesh = plsc.VectorSubcoreMesh(core_axis_name="c", subcore_axis_name="s")
    tok_per_w = n_tok // _NW
    steps = tok_per_w // _CH_TOK

    @functools.partial(
        pl.kernel,
        mesh=mesh,
        out_type=jax.ShapeDtypeStruct((n_tok, D), jnp.float32),
        scratch_types=[
            pltpu.VMEM((_CH,), jnp.int32),
            pltpu.VMEM((_CH,), jnp.int32),
            pltpu.VMEM((_CH, D), jnp.float32),
            pltpu.VMEM((_CH, D), jnp.float32),
            pltpu.VMEM((_CH_TOK, D), jnp.float32),
            pltpu.SemaphoreType.DMA,
            pltpu.SemaphoreType.DMA,
        ],
    )
    def k(table_hbm, idx_hbm, out_hbm, idx_v0, idx_v1, rows_v0, rows_v1,
          out_v, sem0, sem1):
        wid = lax.axis_index("s") * _NC + lax.axis_index("c")
        idx_base = wid * tok_per_w * K
        tok_base = wid * tok_per_w
        bufs = ((idx_v0, rows_v0, sem0), (idx_v1, rows_v1, sem1))

        def start(b, it):
            idx_v, rows_v, sem = bufs[b]
            pltpu.sync_copy(idx_hbm.at[pl.ds(idx_base + it * _CH, _CH)], idx_v)
            pltpu.make_async_copy(table_hbm.at[idx_v], rows_v, sem).start()

        def consume(b, it):
            idx_v, rows_v, sem = bufs[b]
            pltpu.make_async_copy(table_hbm.at[idx_v], rows_v, sem).wait()

            @pl.loop(0, _CH_TOK)
            def _(g):
                @pl.loop(0, D, step=16)
                def _(c):
                    acc = rows_v[K * g, pl.ds(c, 16)]
                    for r in range(1, K):
                        acc = acc + rows_v[K * g + r, pl.ds(c, 16)]
                    out_v[g, pl.ds(c, 16)] = acc

            pltpu.sync_copy(
                out_v, out_hbm.at[pl.ds(tok_base + it * _CH_TOK, _CH_TOK)])

        start(0, 0)

        @pl.loop(0, steps, step=2)
        def _(it):
            start(1, it + 1)
            consume(0, it)

            @pl.when(it + 2 < steps)
            def _():
                start(0, it + 2)

            consume(1, it + 1)

    return k(x_pad, idx_flat)


def _tail_kernel(x_ref, ms_ref, wg_ref, gain_ref, bias_ref, sc_ref, out_ref,
                 *, blk_lo):
    i = pl.program_id(0) + blk_lo
    x = x_ref[...]
    msum = ms_ref[...]
    w_gate = wg_ref[...]
    gain = gain_ref[...]
    bias = bias_ref[...]
    b_gate = sc_ref[0, 0]
    log_mix = sc_ref[0, 1]
    log_scale = sc_ref[0, 2]

    mix = jax.nn.sigmoid(log_mix)
    scale = jax.nn.softplus(log_scale) + jnp.float32(0.01)

    t = i * RB + lax.broadcasted_iota(jnp.int32, (RB, 1), 0)
    deg = jnp.minimum((t + 1).astype(jnp.float32), jnp.float32(K))
    msg = msum / deg

    gate_logit = jnp.sum(x * w_gate, axis=1, keepdims=True) + b_gate
    gate = jax.nn.sigmoid(gate_logit)
    blended = mix * x + (1.0 - mix) * msg
    z = blended * gain + bias
    gelu = 0.5 * z * (1.0 + lax.erf(z * jnp.float32(0.7071067811865476)))
    delta_raw = gelu * scale
    out_ref[...] = gate * delta_raw


def _tail(x2d, msum, w_gate, gain, bias, scalars, blk_lo, n_blk):
    return pl.pallas_call(
        functools.partial(_tail_kernel, blk_lo=blk_lo),
        grid=(n_blk,),
        in_specs=[
            pl.BlockSpec((RB, D), lambda i: (i + blk_lo, 0)),
            pl.BlockSpec((RB, D), lambda i: (i, 0)),
            pl.BlockSpec((1, D), lambda i: (0, 0)),
            pl.BlockSpec((1, D), lambda i: (0, 0)),
            pl.BlockSpec((1, D), lambda i: (0, 0)),
            pl.BlockSpec((1, 4), lambda i: (0, 0)),
        ],
        out_specs=pl.BlockSpec((RB, D), lambda i: (i, 0)),
        out_shape=jax.ShapeDtypeStruct((n_blk * RB, D), jnp.float32),
        compiler_params=pltpu.CompilerParams(
            dimension_semantics=("arbitrary",)),
    )(x2d, msum, w_gate, gain, bias, scalars)


N_CHUNKS = 2
BLK_PER_CHUNK = NB // N_CHUNKS


@jax.jit
def kernel(x, w_gate, b_gate, gain, bias, log_mix, log_scale):
    x2d = x[0]
    x_pad = jnp.concatenate(
        [x2d, jnp.zeros((8, D), jnp.float32)], axis=0)
    scalars = jnp.stack(
        [b_gate, log_mix, log_scale, jnp.float32(0.0)]).reshape(1, 4)
    wg = w_gate.reshape(1, D)
    ga = gain.reshape(1, D)
    bi = bias.reshape(1, D)

    deltas = []
    for c in range(N_CHUNKS):
        blk_lo = c * BLK_PER_CHUNK
        n_tok = BLK_PER_CHUNK * RB
        idx = _topk_indices(x2d, blk_lo, BLK_PER_CHUNK)
        msum = _gather_sum(x_pad, idx.reshape(n_tok * K), n_tok)
        deltas.append(
            _tail(x2d, msum, wg, ga, bi, scalars, blk_lo, BLK_PER_CHUNK))
    return jnp.concatenate(deltas, axis=0)[None]


# speedup vs baseline: 15.1362x; 1.0423x over previous
import functools

import jax
import jax.numpy as jnp
from jax import lax
from jax.experimental import pallas as pl
from jax.experimental.pallas import tpu as pltpu
from jax.experimental.pallas import tpu_sc as plsc

T = 4096
D = 768
K = 8
RB = 256
CB = 256
NB = T // RB
NEG = float(-3.4e38)
INVALID_THRESH = float(-1e30)

_NC = 2
_NS = 16
_NW = _NC * _NS
_CH_TOK = 8
_CH = _CH_TOK * K


def _topk_kernel(x_blk_ref, x_all_ref, idx_out_ref, M_ref, Mi_ref, *, blk_lo):
    i = pl.program_id(0) + blk_lo
    x_r = x_blk_ref[...]
    row_g = i * RB + lax.broadcasted_iota(jnp.int32, (RB, 128), 0)
    lane = lax.broadcasted_iota(jnp.int32, (RB, 128), 1)
    pos_big = jnp.int32(2 ** 30)

    for m in range(K):
        M_ref[m] = jnp.full((RB, 128), NEG, jnp.float32)
        Mi_ref[m] = jnp.zeros((RB, 128), jnp.int32)

    def body(ci, _):
        x_c = x_all_ref[pl.ds(ci * CB, CB), :]
        s = lax.dot_general(
            x_r, x_c, (((1,), (1,)), ((), ())),
            preferred_element_type=jnp.float32)
        for sl in range(CB // 128):
            v = s[:, sl * 128:(sl + 1) * 128]
            col = ci * CB + sl * 128 + lane
            v = jnp.where(col <= row_g, v, NEG)
            ge = [M_ref[m] >= v for m in range(K)]
            for m in range(K - 1, 0, -1):
                M_ref[m] = jnp.where(
                    ge[m], M_ref[m], jnp.where(ge[m - 1], v, M_ref[m - 1]))
                Mi_ref[m] = jnp.where(
                    ge[m], Mi_ref[m], jnp.where(ge[m - 1], col, Mi_ref[m - 1]))
            M_ref[0] = jnp.where(ge[0], M_ref[0], v)
            Mi_ref[0] = jnp.where(ge[0], Mi_ref[0], col)
        return 0

    lax.fori_loop(0, i + 1, body, 0)

    vals_out = []
    idx_out = []
    for _ in range(K):
        top = M_ref[0]
        topi = Mi_ref[0]
        m_val = jnp.max(top, axis=1, keepdims=True)
        m_idx = jnp.min(
            jnp.where(top == m_val, topi, pos_big), axis=1, keepdims=True)
        vals_out.append(m_val)
        idx_out.append(m_idx)
        lanemask = (top == m_val) & (topi == m_idx)
        for m in range(K - 1):
            M_ref[m] = jnp.where(lanemask, M_ref[m + 1], M_ref[m])
            Mi_ref[m] = jnp.where(lanemask, Mi_ref[m + 1], Mi_ref[m])
        M_ref[K - 1] = jnp.where(lanemask, NEG, M_ref[K - 1])

    vals = jnp.concatenate(vals_out, axis=1)
    idx = jnp.concatenate(idx_out, axis=1)
    idx_out_ref[...] = jnp.where(vals <= INVALID_THRESH, jnp.int32(T), idx)


def _topk_indices(x2d, blk_lo, n_blk):
    return pl.pallas_call(
        functools.partial(_topk_kernel, blk_lo=blk_lo),
        grid=(n_blk,),
        in_specs=[
            pl.BlockSpec((RB, D), lambda i: (i + blk_lo, 0)),
            pl.BlockSpec((T, D), lambda i: (0, 0)),
        ],
        out_specs=pl.BlockSpec((RB, K), lambda i: (i, 0)),
        out_shape=jax.ShapeDtypeStruct((n_blk * RB, K), jnp.int32),
        scratch_shapes=[
            pltpu.VMEM((K, RB, 128), jnp.float32),
            pltpu.VMEM((K, RB, 128), jnp.int32),
        ],
        compiler_params=pltpu.CompilerParams(
            dimension_semantics=("arbitrary",)),
    )(x2d, x2d)


def _gather_sum(x_pad, idx_flat, n_tok):
    mesh = plsc.VectorSubcoreMesh(core_axis_name="c", subcore_axis_name="s")
    tok_per_w = n_tok // _NW
    steps = tok_per_w // _CH_TOK

    @functools.partial(
        pl.kernel,
        mesh=mesh,
        out_type=jax.ShapeDtypeStruct((n_tok, D), jnp.float32),
        scratch_types=[
            pltpu.VMEM((_CH,), jnp.int32),
            pltpu.VMEM((_CH,), jnp.int32),
            pltpu.VMEM((_CH, D), jnp.float32),
            pltpu.VMEM((_CH, D), jnp.float32),
            pltpu.VMEM((_CH_TOK, D), jnp.float32),
            pltpu.SemaphoreType.DMA,
            pltpu.SemaphoreType.DMA,
        ],
    )
    def k(table_hbm, idx_hbm, out_hbm, idx_v0, idx_v1, rows_v0, rows_v1,
          out_v, sem0, sem1):
        wid = lax.axis_index("s") * _NC + lax.axis_index("c")
        idx_base = wid * tok_per_w * K
        tok_base = wid * tok_per_w
        bufs = ((idx_v0, rows_v0, sem0), (idx_v1, rows_v1, sem1))

        def start(b, it):
            idx_v, rows_v, sem = bufs[b]
            pltpu.sync_copy(idx_hbm.at[pl.ds(idx_base + it * _CH, _CH)], idx_v)
            pltpu.make_async_copy(table_hbm.at[idx_v], rows_v, sem).start()

        def consume(b, it):
            idx_v, rows_v, sem = bufs[b]
            pltpu.make_async_copy(table_hbm.at[idx_v], rows_v, sem).wait()

            @pl.loop(0, _CH_TOK)
            def _(g):
                @pl.loop(0, D, step=16)
                def _(c):
                    acc = rows_v[K * g, pl.ds(c, 16)]
                    for r in range(1, K):
                        acc = acc + rows_v[K * g + r, pl.ds(c, 16)]
                    out_v[g, pl.ds(c, 16)] = acc

            pltpu.sync_copy(
                out_v, out_hbm.at[pl.ds(tok_base + it * _CH_TOK, _CH_TOK)])

        start(0, 0)

        @pl.loop(0, steps, step=2)
        def _(it):
            start(1, it + 1)
            consume(0, it)

            @pl.when(it + 2 < steps)
            def _():
                start(0, it + 2)

            consume(1, it + 1)

    return k(x_pad, idx_flat)


def _tail_kernel(x_ref, ms_ref, wg_ref, gain_ref, bias_ref, sc_ref, out_ref,
                 *, blk_lo):
    i = pl.program_id(0) + blk_lo
    x = x_ref[...]
    msum = ms_ref[...]
    w_gate = wg_ref[...]
    gain = gain_ref[...]
    bias = bias_ref[...]
    b_gate = sc_ref[0, 0]
    log_mix = sc_ref[0, 1]
    log_scale = sc_ref[0, 2]

    mix = jax.nn.sigmoid(log_mix)
    scale = jax.nn.softplus(log_scale) + jnp.float32(0.01)

    t = i * RB + lax.broadcasted_iota(jnp.int32, (RB, 1), 0)
    deg = jnp.minimum((t + 1).astype(jnp.float32), jnp.float32(K))
    msg = msum / deg

    gate_logit = jnp.sum(x * w_gate, axis=1, keepdims=True) + b_gate
    gate = jax.nn.sigmoid(gate_logit)
    blended = mix * x + (1.0 - mix) * msg
    z = blended * gain + bias
    gelu = 0.5 * z * (1.0 + lax.erf(z * jnp.float32(0.7071067811865476)))
    delta_raw = gelu * scale
    out_ref[...] = gate * delta_raw


def _tail(x2d, msum, w_gate, gain, bias, scalars, blk_lo, n_blk):
    return pl.pallas_call(
        functools.partial(_tail_kernel, blk_lo=blk_lo),
        grid=(n_blk,),
        in_specs=[
            pl.BlockSpec((RB, D), lambda i: (i + blk_lo, 0)),
            pl.BlockSpec((RB, D), lambda i: (i, 0)),
            pl.BlockSpec((1, D), lambda i: (0, 0)),
            pl.BlockSpec((1, D), lambda i: (0, 0)),
            pl.BlockSpec((1, D), lambda i: (0, 0)),
            pl.BlockSpec((1, 4), lambda i: (0, 0)),
        ],
        out_specs=pl.BlockSpec((RB, D), lambda i: (i, 0)),
        out_shape=jax.ShapeDtypeStruct((n_blk * RB, D), jnp.float32),
        compiler_params=pltpu.CompilerParams(
            dimension_semantics=("arbitrary",)),
    )(x2d, msum, w_gate, gain, bias, scalars)


N_CHUNKS = 4
BLK_PER_CHUNK = NB // N_CHUNKS


@jax.jit
def kernel(x, w_gate, b_gate, gain, bias, log_mix, log_scale):
    x2d = x[0]
    x_pad = jnp.concatenate(
        [x2d, jnp.zeros((8, D), jnp.float32)], axis=0)
    scalars = jnp.stack(
        [b_gate, log_mix, log_scale, jnp.float32(0.0)]).reshape(1, 4)
    wg = w_gate.reshape(1, D)
    ga = gain.reshape(1, D)
    bi = bias.reshape(1, D)

    deltas = []
    for c in range(N_CHUNKS):
        blk_lo = c * BLK_PER_CHUNK
        n_tok = BLK_PER_CHUNK * RB
        idx = _topk_indices(x2d, blk_lo, BLK_PER_CHUNK)
        msum = _gather_sum(x_pad, idx.reshape(n_tok * K), n_tok)
        deltas.append(
            _tail(x2d, msum, wg, ga, bi, scalars, blk_lo, BLK_PER_CHUNK))
    return jnp.concatenate(deltas, axis=0)[None]
